# baseline (speedup 1.0000x reference)
import functools

import jax
import jax.numpy as jnp
from jax import lax
from jax.experimental import pallas as pl
from jax.experimental.pallas import tpu as pltpu
from jax.experimental.pallas import tpu_sc as plsc

N = 10000
E = 320000
D = 128
H = 128
O = 64
EPS = 1e-5

NC = 2
NS = 16
CHUNK = 128
RPT = 160
EPAD = NS * RPT * CHUNK
NPAD = 10240
STRIPE = NPAD // NS
CPP = 40
C0ROWS = 144
C1ROWS = RPT - C0ROWS


def _pass_plan(rows):
    plan, off = [], 0
    while rows > 0:
        take = min(CPP, rows)
        plan.append((off, take))
        off += take
        rows -= take
    return plan

BR = 2000
GRID = N // BR


def _dott(a, w):
    return lax.dot_general(a, w, (((1,), (1,)), ((), ())),
                           preferred_element_type=jnp.float32)


def _make_agg(with_count: bool):
    mesh = plsc.VectorSubcoreMesh(core_axis_name="c", subcore_axis_name="s")
    out_type = [jax.ShapeDtypeStruct((NC, NPAD, H), jnp.float32)]
    if with_count:
        out_type.append(jax.ShapeDtypeStruct((NC, NPAD), jnp.float32))
    scratch = [
        pltpu.VMEM((CPP, CHUNK), jnp.int32),
        pltpu.VMEM((CPP, CHUNK), jnp.int32),
        pltpu.VMEM((CHUNK, H), jnp.float32),
        pltpu.VMEM((CHUNK, H), jnp.float32),
        pltpu.VMEM((CHUNK,), jnp.float32),
        pltpu.VMEM_SHARED((NPAD, H), jnp.float32),
        pltpu.VMEM_SHARED((NPAD,), jnp.float32),
        pltpu.SemaphoreType.DMA,
        pltpu.SemaphoreType.DMA,
        pltpu.SemaphoreType.DMA,
        pltpu.SemaphoreType.DMA,
        pltpu.SemaphoreType.DMA,
        pltpu.SemaphoreType.DMA,
    ]

    @functools.partial(pl.kernel, mesh=mesh, out_type=out_type,
                       scratch_types=scratch)
    def agg(h_hbm, src_hbm, dst_hbm, *rest):
        if with_count:
            (part_out, cnt_out, src_v, dst_v, rows_a, rows_b, ones_v,
             acc_s, cnt_s, sem_ga, sem_gb, sem_sa, sem_sb, sem_ca,
             sem_cb) = rest
        else:
            (part_out, src_v, dst_v, rows_a, rows_b, ones_v,
             acc_s, cnt_s, sem_ga, sem_gb, sem_sa, sem_sb, sem_ca,
             sem_cb) = rest
        cid = lax.axis_index("c")
        sid = lax.axis_index("s")

        def zrow(r, carry):
            for k in range(CHUNK // 16):
                rows_a[r, pl.ds(k * 16, 16)] = jnp.zeros((16,), jnp.float32)
            return carry
        lax.fori_loop(0, CHUNK, zrow, 0)
        for q in range(STRIPE // CHUNK):
            pltpu.sync_copy(
                rows_a, acc_s.at[pl.ds(sid * STRIPE + q * CHUNK, CHUNK)])
        if with_count:
            for k in range(CHUNK // 16):
                ones_v[pl.ds(k * 16, 16)] = jnp.full((16,), 1.0, jnp.float32)
            def zc(r, carry):
                pltpu.sync_copy(
                    rows_a.at[0, pl.ds(0, CHUNK)],
                    cnt_s.at[pl.ds(sid * STRIPE + r * CHUNK, CHUNK)])
                return carry
            lax.fori_loop(0, STRIPE // CHUNK, zc, 0)
        plsc.subcore_barrier()

        def _scat(buf, j, sem_s, sem_c):
            pltpu.async_copy(buf, acc_s.at[dst_v.at[j]], sem_s, add=True)
            if with_count:
                pltpu.async_copy(ones_v, cnt_s.at[dst_v.at[j]], sem_c,
                                 add=True)

        def _scat_wait(buf, sem_s, sem_c):
            pltpu.make_async_copy(buf, acc_s.at[dst_v.at[0]], sem_s).wait()
            if with_count:
                pltpu.make_async_copy(ones_v, cnt_s.at[dst_v.at[0]],
                                      sem_c).wait()

        def _pass(base, n):
            pltpu.sync_copy(src_hbm.at[pl.ds(base, n)],
                            src_v.at[pl.ds(0, n)])
            pltpu.sync_copy(dst_hbm.at[pl.ds(base, n)],
                            dst_v.at[pl.ds(0, n)])

            pltpu.async_copy(h_hbm.at[src_v.at[0]], rows_a, sem_ga).wait()
            _scat(rows_a, 0, sem_sa, sem_ca)
            pltpu.async_copy(h_hbm.at[src_v.at[1]], rows_b, sem_gb)

            def body(i, carry):
                j1 = 2 * i + 1
                pltpu.make_async_copy(h_hbm.at[src_v.at[j1]], rows_b,
                                      sem_gb).wait()
                _scat(rows_b, j1, sem_sb, sem_cb)
                _scat_wait(rows_a, sem_sa, sem_ca)
                pltpu.async_copy(h_hbm.at[src_v.at[j1 + 1]], rows_a, sem_ga)
                pltpu.make_async_copy(h_hbm.at[src_v.at[j1 + 1]], rows_a,
                                      sem_ga).wait()
                _scat(rows_a, j1 + 1, sem_sa, sem_ca)
                _scat_wait(rows_b, sem_sb, sem_cb)
                pltpu.async_copy(h_hbm.at[src_v.at[j1 + 2]], rows_b, sem_gb)
                return carry

            lax.fori_loop(0, n // 2 - 1, body, 0)
            pltpu.make_async_copy(h_hbm.at[src_v.at[n - 1]], rows_b,
                                  sem_gb).wait()
            _scat(rows_b, n - 1, sem_sb, sem_cb)
            _scat_wait(rows_a, sem_sa, sem_ca)
            _scat_wait(rows_b, sem_sb, sem_cb)

        @pl.when(cid == 0)
        def _run0():
            for off, take in _pass_plan(C0ROWS):
                _pass(sid * C0ROWS + off, take)

        @pl.when(cid == 1)
        def _run1():
            for off, take in _pass_plan(C1ROWS):
                _pass(NS * C0ROWS + sid * C1ROWS + off, take)

        plsc.subcore_barrier()

        pltpu.sync_copy(acc_s.at[pl.ds(sid * STRIPE, STRIPE)],
                        part_out.at[cid, pl.ds(sid * STRIPE, STRIPE)])
        if with_count:
            pltpu.sync_copy(cnt_s.at[pl.ds(sid * STRIPE, STRIPE)],
                            cnt_out.at[cid, pl.ds(sid * STRIPE, STRIPE)])

    return agg


def _tc_proj(x, wp, b, s, t):
    def body(x_ref, w_ref, b_ref, s_ref, t_ref, o_ref):
        h = jnp.maximum(_dott(x_ref[...], w_ref[...]) + b_ref[...], 0.0)
        o_ref[...] = h * s_ref[...] + t_ref[...]

    return pl.pallas_call(
        body,
        grid=(GRID,),
        in_specs=[
            pl.BlockSpec((BR, D), lambda i: (i, 0)),
            pl.BlockSpec((H, D), lambda i: (0, 0)),
            pl.BlockSpec((1, H), lambda i: (0, 0)),
            pl.BlockSpec((1, H), lambda i: (0, 0)),
            pl.BlockSpec((1, H), lambda i: (0, 0)),
        ],
        out_specs=pl.BlockSpec((BR, H), lambda i: (i, 0)),
        out_shape=jax.ShapeDtypeStruct((N, H), jnp.float32),
    )(x, wp, b, s, t)


def _tc_combine(parts, cnt_t, h, wl, bl, wr):
    def body(p_ref, c_ref, h_ref, wl_ref, bl_ref, wr_ref, o_ref):
        p = p_ref[0] + p_ref[1]
        c = c_ref[:, 0:1] + c_ref[:, 1:2]
        inv = 1.0 / jnp.maximum(c, 1.0)
        mean = p * inv
        o_ref[...] = jnp.maximum(
            _dott(mean, wl_ref[...]) + bl_ref[...] + _dott(h_ref[...], wr_ref[...]),
            0.0)

    return pl.pallas_call(
        body,
        grid=(GRID,),
        in_specs=[
            pl.BlockSpec((NC, BR, H), lambda i: (0, i, 0)),
            pl.BlockSpec((BR, NC), lambda i: (i, 0)),
            pl.BlockSpec((BR, H), lambda i: (i, 0)),
            pl.BlockSpec((H, H), lambda i: (0, 0)),
            pl.BlockSpec((1, H), lambda i: (0, 0)),
            pl.BlockSpec((H, H), lambda i: (0, 0)),
        ],
        out_specs=pl.BlockSpec((BR, H), lambda i: (i, 0)),
        out_shape=jax.ShapeDtypeStruct((N, H), jnp.float32),
    )(parts, cnt_t, h, wl, bl, wr)


def _tc_final(parts, cnt_t, h1, hloc, wl, bl, wr, wfa, wfb, bf, s2, t2, wc, bc):
    def body(p_ref, c_ref, h1_ref, hl_ref, wl_ref, bl_ref, wr_ref,
             wfa_ref, wfb_ref, bf_ref, s_ref, t_ref, wc_ref, bc_ref,
             z_ref, lg_ref):
        p = p_ref[0] + p_ref[1]
        c = c_ref[:, 0:1] + c_ref[:, 1:2]
        inv = 1.0 / jnp.maximum(c, 1.0)
        mean = p * inv
        h2 = jnp.maximum(
            _dott(mean, wl_ref[...]) + bl_ref[...] + _dott(h1_ref[...], wr_ref[...]),
            0.0)
        zp = _dott(hl_ref[...], wfa_ref[...]) + _dott(h2, wfb_ref[...]) + bf_ref[...]
        z = jnp.maximum(zp, 0.0) * s_ref[...] + t_ref[...]
        z_ref[...] = z
        lg_ref[...] = jnp.sum(z * wc_ref[...], axis=1, keepdims=True) + bc_ref[0, 0]

    return pl.pallas_call(
        body,
        grid=(GRID,),
        in_specs=[
            pl.BlockSpec((NC, BR, H), lambda i: (0, i, 0)),
            pl.BlockSpec((BR, NC), lambda i: (i, 0)),
            pl.BlockSpec((BR, H), lambda i: (i, 0)),
            pl.BlockSpec((BR, H), lambda i: (i, 0)),
            pl.BlockSpec((H, H), lambda i: (0, 0)),
            pl.BlockSpec((1, H), lambda i: (0, 0)),
            pl.BlockSpec((H, H), lambda i: (0, 0)),
            pl.BlockSpec((O, H), lambda i: (0, 0)),
            pl.BlockSpec((O, H), lambda i: (0, 0)),
            pl.BlockSpec((1, O), lambda i: (0, 0)),
            pl.BlockSpec((1, O), lambda i: (0, 0)),
            pl.BlockSpec((1, O), lambda i: (0, 0)),
            pl.BlockSpec((1, O), lambda i: (0, 0)),
            pl.BlockSpec((1, 1), lambda i: (0, 0)),
        ],
        out_specs=[
            pl.BlockSpec((BR, O), lambda i: (i, 0)),
            pl.BlockSpec((BR, 1), lambda i: (i, 0)),
        ],
        out_shape=[
            jax.ShapeDtypeStruct((N, O), jnp.float32),
            jax.ShapeDtypeStruct((N, 1), jnp.float32),
        ],
    )(parts, cnt_t, h1, hloc, wl, bl, wr, wfa, wfb, bf, s2, t2, wc, bc)


def kernel(x, edge_index, W_proj, b_proj, bn1_g, bn1_b, Wl0, bl0, Wr0,
           Wl1, bl1, Wr1, W_fus, b_fus, bn2_g, bn2_b, W_cls, b_cls):
    ei = edge_index.astype(jnp.int32)
    npad_e = EPAD - E
    src2d = jnp.concatenate(
        [ei[0], jnp.zeros((npad_e,), jnp.int32)]).reshape(NS * RPT, CHUNK)
    dst2d = jnp.concatenate(
        [ei[1], jnp.full((npad_e,), N, jnp.int32)]).reshape(NS * RPT, CHUNK)
    bn_s = 1.0 / jnp.sqrt(1.0 + EPS)
    s1 = (bn1_g * bn_s).reshape(1, H)
    t1 = bn1_b.reshape(1, H)
    s2 = (bn2_g * bn_s).reshape(1, O)
    t2 = bn2_b.reshape(1, O)

    h_local = _tc_proj(x, W_proj, b_proj.reshape(1, H), s1, t1)

    parts0, cnt2 = _make_agg(True)(h_local, src2d, dst2d)
    cnt_t = cnt2.T

    h1 = _tc_combine(parts0, cnt_t, h_local, Wl0, bl0.reshape(1, H), Wr0)

    parts1, _ = _make_agg(True)(h1, src2d, dst2d)

    z, lg = _tc_final(parts1, cnt_t, h1, h_local, Wl1, bl1.reshape(1, H), Wr1,
                      W_fus[:, :H], W_fus[:, H:], b_fus.reshape(1, O),
                      s2, t2, W_cls, b_cls.reshape(1, 1))
    return (lg.reshape(-1), z)


# speedup vs baseline: 1.0219x; 1.0219x over previous
import functools

import jax
import jax.numpy as jnp
from jax import lax
from jax.experimental import pallas as pl
from jax.experimental.pallas import tpu as pltpu
from jax.experimental.pallas import tpu_sc as plsc

N = 10000
E = 320000
D = 128
H = 128
O = 64
EPS = 1e-5

NC = 2
NS = 16
CHUNK = 128
RPT = 160
EPAD = NS * RPT * CHUNK
NPAD = 10240
STRIPE = NPAD // NS
CPP = 24
C0ROWS = 144
C1ROWS = RPT - C0ROWS


def _pass_plan(rows):
    plan, off = [], 0
    while rows > 0:
        take = min(CPP, rows)
        plan.append((off, take))
        off += take
        rows -= take
    return plan

BR = 2000
GRID = N // BR


def _dott(a, w):
    return lax.dot_general(a, w, (((1,), (1,)), ((), ())),
                           preferred_element_type=jnp.float32)


def _make_agg(with_count: bool):
    mesh = plsc.VectorSubcoreMesh(core_axis_name="c", subcore_axis_name="s")
    out_type = [jax.ShapeDtypeStruct((NC, NPAD, H), jnp.float32)]
    if with_count:
        out_type.append(jax.ShapeDtypeStruct((NC, NPAD), jnp.float32))
    scratch = [
        pltpu.VMEM((2, CPP, CHUNK), jnp.int32),
        pltpu.VMEM((2, CPP, CHUNK), jnp.int32),
        pltpu.VMEM((CHUNK, H), jnp.float32),
        pltpu.VMEM((CHUNK, H), jnp.float32),
        pltpu.VMEM((CHUNK,), jnp.float32),
        pltpu.VMEM_SHARED((NPAD, H), jnp.float32),
        pltpu.VMEM_SHARED((NPAD,), jnp.float32),
        pltpu.SemaphoreType.DMA,
        pltpu.SemaphoreType.DMA,
        pltpu.SemaphoreType.DMA,
        pltpu.SemaphoreType.DMA,
        pltpu.SemaphoreType.DMA,
        pltpu.SemaphoreType.DMA,
        pltpu.SemaphoreType.DMA,
    ]

    @functools.partial(pl.kernel, mesh=mesh, out_type=out_type,
                       scratch_types=scratch)
    def agg(h_hbm, src_hbm, dst_hbm, *rest):
        if with_count:
            (part_out, cnt_out, src_v, dst_v, rows_a, rows_b, ones_v,
             acc_s, cnt_s, sem_ga, sem_gb, sem_sa, sem_sb, sem_ca,
             sem_cb, sem_ix) = rest
        else:
            (part_out, src_v, dst_v, rows_a, rows_b, ones_v,
             acc_s, cnt_s, sem_ga, sem_gb, sem_sa, sem_sb, sem_ca,
             sem_cb, sem_ix) = rest
        cid = lax.axis_index("c")
        sid = lax.axis_index("s")

        def zrow(r, carry):
            for k in range(CHUNK // 16):
                rows_a[r, pl.ds(k * 16, 16)] = jnp.zeros((16,), jnp.float32)
            return carry
        lax.fori_loop(0, CHUNK, zrow, 0)
        for q in range(STRIPE // CHUNK):
            pltpu.sync_copy(
                rows_a, acc_s.at[pl.ds(sid * STRIPE + q * CHUNK, CHUNK)])
        if with_count:
            for k in range(CHUNK // 16):
                ones_v[pl.ds(k * 16, 16)] = jnp.full((16,), 1.0, jnp.float32)
            def zc(r, carry):
                pltpu.sync_copy(
                    rows_a.at[0, pl.ds(0, CHUNK)],
                    cnt_s.at[pl.ds(sid * STRIPE + r * CHUNK, CHUNK)])
                return carry
            lax.fori_loop(0, STRIPE // CHUNK, zc, 0)
        plsc.subcore_barrier()

        def _stage(base, n, slot):
            pltpu.async_copy(src_hbm.at[pl.ds(base, n)],
                             src_v.at[slot, pl.ds(0, n)], sem_ix)
            pltpu.async_copy(dst_hbm.at[pl.ds(base, n)],
                             dst_v.at[slot, pl.ds(0, n)], sem_ix)

        def _stage_wait(base, n, slot):
            pltpu.make_async_copy(src_hbm.at[pl.ds(base, n)],
                                  src_v.at[slot, pl.ds(0, n)], sem_ix).wait()
            pltpu.make_async_copy(dst_hbm.at[pl.ds(base, n)],
                                  dst_v.at[slot, pl.ds(0, n)], sem_ix).wait()

        def _pass(n, slot):
            sv = src_v.at[slot]
            dv = dst_v.at[slot]

            def _scat(buf, j, sem_s, sem_c):
                pltpu.async_copy(buf, acc_s.at[dv.at[j]], sem_s, add=True)
                if with_count:
                    pltpu.async_copy(ones_v, cnt_s.at[dv.at[j]], sem_c,
                                     add=True)

            def _scat_wait(buf, sem_s, sem_c):
                pltpu.make_async_copy(buf, acc_s.at[dv.at[0]], sem_s).wait()
                if with_count:
                    pltpu.make_async_copy(ones_v, cnt_s.at[dv.at[0]],
                                          sem_c).wait()

            pltpu.async_copy(h_hbm.at[sv.at[0]], rows_a, sem_ga).wait()
            _scat(rows_a, 0, sem_sa, sem_ca)
            pltpu.async_copy(h_hbm.at[sv.at[1]], rows_b, sem_gb)

            def body(i, carry):
                j1 = 2 * i + 1
                pltpu.make_async_copy(h_hbm.at[sv.at[j1]], rows_b,
                                      sem_gb).wait()
                _scat(rows_b, j1, sem_sb, sem_cb)
                _scat_wait(rows_a, sem_sa, sem_ca)
                pltpu.async_copy(h_hbm.at[sv.at[j1 + 1]], rows_a, sem_ga)
                pltpu.make_async_copy(h_hbm.at[sv.at[j1 + 1]], rows_a,
                                      sem_ga).wait()
                _scat(rows_a, j1 + 1, sem_sa, sem_ca)
                _scat_wait(rows_b, sem_sb, sem_cb)
                pltpu.async_copy(h_hbm.at[sv.at[j1 + 2]], rows_b, sem_gb)
                return carry

            lax.fori_loop(0, n // 2 - 1, body, 0)
            pltpu.make_async_copy(h_hbm.at[sv.at[n - 1]], rows_b,
                                  sem_gb).wait()
            _scat(rows_b, n - 1, sem_sb, sem_cb)
            _scat_wait(rows_a, sem_sa, sem_ca)
            _scat_wait(rows_b, sem_sb, sem_cb)

        def _run(plan, tile_base):
            _stage(tile_base + plan[0][0], plan[0][1], 0)
            for i, (off, take) in enumerate(plan):
                slot = i % 2
                _stage_wait(tile_base + off, take, slot)
                if i + 1 < len(plan):
                    noff, ntake = plan[i + 1]
                    _stage(tile_base + noff, ntake, 1 - slot)
                _pass(take, slot)

        @pl.when(cid == 0)
        def _run0():
            _run(_pass_plan(C0ROWS), sid * C0ROWS)

        @pl.when(cid == 1)
        def _run1():
            _run(_pass_plan(C1ROWS), NS * C0ROWS + sid * C1ROWS)

        plsc.subcore_barrier()

        pltpu.sync_copy(acc_s.at[pl.ds(sid * STRIPE, STRIPE)],
                        part_out.at[cid, pl.ds(sid * STRIPE, STRIPE)])
        if with_count:
            pltpu.sync_copy(cnt_s.at[pl.ds(sid * STRIPE, STRIPE)],
                            cnt_out.at[cid, pl.ds(sid * STRIPE, STRIPE)])

    return agg


def _tc_proj(x, wp, b, s, t):
    def body(x_ref, w_ref, b_ref, s_ref, t_ref, o_ref):
        h = jnp.maximum(_dott(x_ref[...], w_ref[...]) + b_ref[...], 0.0)
        o_ref[...] = h * s_ref[...] + t_ref[...]

    return pl.pallas_call(
        body,
        grid=(GRID,),
        in_specs=[
            pl.BlockSpec((BR, D), lambda i: (i, 0)),
            pl.BlockSpec((H, D), lambda i: (0, 0)),
            pl.BlockSpec((1, H), lambda i: (0, 0)),
            pl.BlockSpec((1, H), lambda i: (0, 0)),
            pl.BlockSpec((1, H), lambda i: (0, 0)),
        ],
        out_specs=pl.BlockSpec((BR, H), lambda i: (i, 0)),
        out_shape=jax.ShapeDtypeStruct((N, H), jnp.float32),
    )(x, wp, b, s, t)


def _tc_combine(parts, cnt_t, h, wl, bl, wr):
    def body(p_ref, c_ref, h_ref, wl_ref, bl_ref, wr_ref, o_ref):
        p = p_ref[0] + p_ref[1]
        c = c_ref[:, 0:1] + c_ref[:, 1:2]
        inv = 1.0 / jnp.maximum(c, 1.0)
        mean = p * inv
        o_ref[...] = jnp.maximum(
            _dott(mean, wl_ref[...]) + bl_ref[...] + _dott(h_ref[...], wr_ref[...]),
            0.0)

    return pl.pallas_call(
        body,
        grid=(GRID,),
        in_specs=[
            pl.BlockSpec((NC, BR, H), lambda i: (0, i, 0)),
            pl.BlockSpec((BR, NC), lambda i: (i, 0)),
            pl.BlockSpec((BR, H), lambda i: (i, 0)),
            pl.BlockSpec((H, H), lambda i: (0, 0)),
            pl.BlockSpec((1, H), lambda i: (0, 0)),
            pl.BlockSpec((H, H), lambda i: (0, 0)),
        ],
        out_specs=pl.BlockSpec((BR, H), lambda i: (i, 0)),
        out_shape=jax.ShapeDtypeStruct((N, H), jnp.float32),
    )(parts, cnt_t, h, wl, bl, wr)


def _tc_final(parts, cnt_t, h1, hloc, wl, bl, wr, wfa, wfb, bf, s2, t2, wc, bc):
    def body(p_ref, c_ref, h1_ref, hl_ref, wl_ref, bl_ref, wr_ref,
             wfa_ref, wfb_ref, bf_ref, s_ref, t_ref, wc_ref, bc_ref,
             z_ref, lg_ref):
        p = p_ref[0] + p_ref[1]
        c = c_ref[:, 0:1] + c_ref[:, 1:2]
        inv = 1.0 / jnp.maximum(c, 1.0)
        mean = p * inv
        h2 = jnp.maximum(
            _dott(mean, wl_ref[...]) + bl_ref[...] + _dott(h1_ref[...], wr_ref[...]),
            0.0)
        zp = _dott(hl_ref[...], wfa_ref[...]) + _dott(h2, wfb_ref[...]) + bf_ref[...]
        z = jnp.maximum(zp, 0.0) * s_ref[...] + t_ref[...]
        z_ref[...] = z
        lg_ref[...] = jnp.sum(z * wc_ref[...], axis=1, keepdims=True) + bc_ref[0, 0]

    return pl.pallas_call(
        body,
        grid=(GRID,),
        in_specs=[
            pl.BlockSpec((NC, BR, H), lambda i: (0, i, 0)),
            pl.BlockSpec((BR, NC), lambda i: (i, 0)),
            pl.BlockSpec((BR, H), lambda i: (i, 0)),
            pl.BlockSpec((BR, H), lambda i: (i, 0)),
            pl.BlockSpec((H, H), lambda i: (0, 0)),
            pl.BlockSpec((1, H), lambda i: (0, 0)),
            pl.BlockSpec((H, H), lambda i: (0, 0)),
            pl.BlockSpec((O, H), lambda i: (0, 0)),
            pl.BlockSpec((O, H), lambda i: (0, 0)),
            pl.BlockSpec((1, O), lambda i: (0, 0)),
            pl.BlockSpec((1, O), lambda i: (0, 0)),
            pl.BlockSpec((1, O), lambda i: (0, 0)),
            pl.BlockSpec((1, O), lambda i: (0, 0)),
            pl.BlockSpec((1, 1), lambda i: (0, 0)),
        ],
        out_specs=[
            pl.BlockSpec((BR, O), lambda i: (i, 0)),
            pl.BlockSpec((BR, 1), lambda i: (i, 0)),
        ],
        out_shape=[
            jax.ShapeDtypeStruct((N, O), jnp.float32),
            jax.ShapeDtypeStruct((N, 1), jnp.float32),
        ],
    )(parts, cnt_t, h1, hloc, wl, bl, wr, wfa, wfb, bf, s2, t2, wc, bc)


def kernel(x, edge_index, W_proj, b_proj, bn1_g, bn1_b, Wl0, bl0, Wr0,
           Wl1, bl1, Wr1, W_fus, b_fus, bn2_g, bn2_b, W_cls, b_cls):
    ei = edge_index.astype(jnp.int32)
    npad_e = EPAD - E
    src2d = jnp.concatenate(
        [ei[0], jnp.zeros((npad_e,), jnp.int32)]).reshape(NS * RPT, CHUNK)
    dst2d = jnp.concatenate(
        [ei[1], jnp.full((npad_e,), N, jnp.int32)]).reshape(NS * RPT, CHUNK)
    bn_s = 1.0 / jnp.sqrt(1.0 + EPS)
    s1 = (bn1_g * bn_s).reshape(1, H)
    t1 = bn1_b.reshape(1, H)
    s2 = (bn2_g * bn_s).reshape(1, O)
    t2 = bn2_b.reshape(1, O)

    h_local = _tc_proj(x, W_proj, b_proj.reshape(1, H), s1, t1)

    parts0, cnt2 = _make_agg(True)(h_local, src2d, dst2d)
    cnt_t = cnt2.T

    h1 = _tc_combine(parts0, cnt_t, h_local, Wl0, bl0.reshape(1, H), Wr0)

    (parts1,) = _make_agg(False)(h1, src2d, dst2d)

    z, lg = _tc_final(parts1, cnt_t, h1, h_local, Wl1, bl1.reshape(1, H), Wr1,
                      W_fus[:, :H], W_fus[:, H:], b_fus.reshape(1, O),
                      s2, t2, W_cls, b_cls.reshape(1, 1))
    return (lg.reshape(-1), z)
